# baseline (speedup 1.0000x reference)
import functools

import jax
import jax.numpy as jnp
from jax import lax
from jax.experimental import pallas as pl
from jax.experimental.pallas import tpu as pltpu
from jax.experimental.pallas import tpu_sc as plsc

B = 16384
D = 16


def _sc_gather4(uidx, iidx, ueg, ieg, uem, iem):
    info = plsc.get_sparse_core_info()
    nw = info.num_cores * info.num_subcores
    bpw = B // nw
    mesh = plsc.VectorSubcoreMesh(core_axis_name="c", subcore_axis_name="s")

    @functools.partial(
        pl.kernel,
        mesh=mesh,
        out_type=[jax.ShapeDtypeStruct((B, D), jnp.float32)] * 4,
        scratch_types=[
            pltpu.VMEM((bpw,), jnp.int32),
            pltpu.VMEM((bpw,), jnp.int32),
            pltpu.VMEM((bpw, D), jnp.float32),
            pltpu.VMEM((bpw, D), jnp.float32),
            pltpu.VMEM((bpw, D), jnp.float32),
            pltpu.VMEM((bpw, D), jnp.float32),
            pltpu.SemaphoreType.DMA,
        ],
        compiler_params=pltpu.CompilerParams(use_tc_tiling_on_sc=False),
    )
    def k(uidx_hbm, iidx_hbm, ueg_hbm, ieg_hbm, uem_hbm, iem_hbm,
          oug, oig, oum, oim, uv, iv, r0, r1, r2, r3, sem):
        wid = lax.axis_index("s") * info.num_cores + lax.axis_index("c")
        base = wid * bpw
        pltpu.sync_copy(uidx_hbm.at[pl.ds(base, bpw)], uv)
        pltpu.sync_copy(iidx_hbm.at[pl.ds(base, bpw)], iv)
        c0 = pltpu.async_copy(ueg_hbm.at[uv], r0, sem)
        c1 = pltpu.async_copy(ieg_hbm.at[iv], r1, sem)
        c2 = pltpu.async_copy(uem_hbm.at[uv], r2, sem)
        c3 = pltpu.async_copy(iem_hbm.at[iv], r3, sem)
        c0.wait()
        c1.wait()
        c2.wait()
        c3.wait()
        pltpu.sync_copy(r0, oug.at[pl.ds(base, bpw)])
        pltpu.sync_copy(r1, oig.at[pl.ds(base, bpw)])
        pltpu.sync_copy(r2, oum.at[pl.ds(base, bpw)])
        pltpu.sync_copy(r3, oim.at[pl.ds(base, bpw)])

    return k(uidx, iidx, ueg, ieg, uem, iem)


def _tc_mlp_body(ug_r, ig_r, um_r, im_r, w0a_r, w0b_r, b0_r, w1_r, b1_r,
                 w2_r, b2_r, w3_r, b3_r, wpg_r, wph_r, bp_r, out_r):
    f32 = jnp.float32
    gmf = ug_r[...] * ig_r[...]
    h = jnp.dot(um_r[...], w0a_r[...], preferred_element_type=f32)
    h = h + jnp.dot(im_r[...], w0b_r[...], preferred_element_type=f32)
    h = jnp.maximum(h + b0_r[...], 0.0)
    h = jnp.maximum(jnp.dot(h, w1_r[...], preferred_element_type=f32) + b1_r[...], 0.0)
    h = jnp.maximum(jnp.dot(h, w2_r[...], preferred_element_type=f32) + b2_r[...], 0.0)
    h = jnp.maximum(jnp.dot(h, w3_r[...], preferred_element_type=f32) + b3_r[...], 0.0)
    pred = jnp.dot(gmf, wpg_r[...], preferred_element_type=f32)
    pred = pred + jnp.dot(h, wph_r[...], preferred_element_type=f32)
    out_r[...] = pred + bp_r[...]


def _tc_mlp(ug, ig, um, im, w0a, w0b, b0, w1t, b1, w2t, b2, w3t, b3,
            wpg, wph, bp2):
    nblk = 8
    rb = B // nblk
    row_spec = pl.BlockSpec((rb, D), lambda i: (i, 0))

    def full(x):
        return pl.BlockSpec(x.shape, lambda i: (0,) * x.ndim)

    return pl.pallas_call(
        _tc_mlp_body,
        grid=(nblk,),
        in_specs=[row_spec, row_spec, row_spec, row_spec,
                  full(w0a), full(w0b), full(b0), full(w1t), full(b1),
                  full(w2t), full(b2), full(w3t), full(b3),
                  full(wpg), full(wph), full(bp2)],
        out_specs=pl.BlockSpec((rb, 1), lambda i: (i, 0)),
        out_shape=jax.ShapeDtypeStruct((B, 1), jnp.float32),
    )(ug, ig, um, im, w0a, w0b, b0, w1t, b1, w2t, b2, w3t, b3, wpg, wph, bp2)


def kernel(user_indices, item_indices, user_embed_gmf, item_embed_gmf,
           user_embed_mlp, item_embed_mlp,
           W0, b0, W1, b1, W2, b2, W3, b3, Wp, bp):
    uidx = user_indices.astype(jnp.int32)
    iidx = item_indices.astype(jnp.int32)
    ug, ig, um, im = _sc_gather4(uidx, iidx, user_embed_gmf, item_embed_gmf,
                                 user_embed_mlp, item_embed_mlp)
    w0a = W0[:, :D].T
    w0b = W0[:, D:].T
    wpg = Wp[:, :D].T
    wph = Wp[:, D:].T
    pred = _tc_mlp(ug, ig, um, im, w0a, w0b, b0.reshape(1, -1),
                   W1.T, b1.reshape(1, -1), W2.T, b2.reshape(1, -1),
                   W3.T, b3.reshape(1, -1), wpg, wph, bp.reshape(1, 1))
    return jnp.squeeze(pred, axis=-1)
